# Initial kernel scaffold; baseline (speedup 1.0000x reference)
import jax
import jax.numpy as jnp
from jax.experimental import pallas as pl


def kernel(croutes, tailcs, rc_cid_emb, rc_weight):
    raise NotImplementedError("write your pallas kernel here")


# speedup vs baseline: 1.1218x; 1.1218x over previous
import functools

import jax
import jax.numpy as jnp
from jax import lax
from jax.experimental import pallas as pl
from jax.experimental.pallas import tpu as pltpu
from jax.experimental.pallas import tpu_sc as plsc

_B, _S, _LVL, _EMB = 1024, 50, 8, 64
_NTOK = _B * _S
_LANES = 16
_CHUNK = 80
_CROWS = _CHUNK * _LVL
_IDXR = _CROWS // 128


def _sc_body(idx_hbm, w_hbm, table_hbm, out_hbm, idx_v, rows_v, out_v, wv, sem):
    info = plsc.get_sparse_core_info()
    nc, ns = info.num_cores, info.num_subcores
    nw = nc * ns
    tpw = _NTOK // nw
    nchunk = tpw // _CHUNK
    wid = lax.axis_index("s") * nc + lax.axis_index("c")
    out_row0 = wid * tpw

    pltpu.sync_copy(w_hbm, wv)
    w = wv[...]
    ws = [w[l] for l in range(_LVL)]
    m = ws[0]
    for l in range(1, _LVL):
        m = jnp.maximum(m, ws[l])
    e = jnp.exp(w - m)
    es = [e[l] for l in range(_LVL)]
    s = es[0]
    for l in range(1, _LVL):
        s = s + es[l]
    alpha = e / s
    a = [alpha[l] for l in range(_LVL)]

    def chunk_body(g, carry):
        pltpu.sync_copy(idx_hbm.at[wid, g], idx_v)
        copies = [
            pltpu.async_copy(
                table_hbm.at[idx_v.at[j]],
                rows_v.at[pl.ds(j * 128, 128)],
                sem,
            )
            for j in range(_IDXR)
        ]
        for c in copies:
            c.wait()

        def tok_body(t, tcarry):
            rbase = t * _LVL
            for j in range(_EMB // _LANES):
                sl = pl.ds(j * _LANES, _LANES)
                acc = a[0] * rows_v[rbase, sl]
                for l in range(1, _LVL):
                    acc = acc + a[l] * rows_v[rbase + l, sl]
                out_v[t, sl] = acc
            return tcarry

        lax.fori_loop(0, _CHUNK, tok_body, 0)
        pltpu.sync_copy(out_v, out_hbm.at[pl.ds(out_row0 + g * _CHUNK, _CHUNK)])
        return carry

    lax.fori_loop(0, nchunk, chunk_body, 0)


@functools.partial(jax.jit, static_argnums=())
def _sc_gather_combine(idx, w_pad, table):
    run = functools.partial(
        pl.kernel,
        out_type=jax.ShapeDtypeStruct((_NTOK, _EMB), jnp.float32),
        mesh=plsc.VectorSubcoreMesh(core_axis_name="c", subcore_axis_name="s"),
        scratch_types=[
            pltpu.VMEM((_IDXR, 128), jnp.int32),
            pltpu.VMEM((_CROWS, _EMB), jnp.float32),
            pltpu.VMEM((_CHUNK, _EMB), jnp.float32),
            pltpu.VMEM((_LANES,), jnp.float32),
            pltpu.SemaphoreType.DMA,
        ],
        compiler_params=pltpu.CompilerParams(use_tc_tiling_on_sc=False),
    )(_sc_body)
    return run(idx, w_pad, table)


def kernel(croutes, tailcs, rc_cid_emb, rc_weight):
    del tailcs
    idx = croutes.reshape(32, _NTOK // 32 // _CHUNK, _IDXR, 128)
    w_pad = jnp.concatenate(
        [rc_weight.astype(jnp.float32),
         jnp.full((_LANES - _LVL,), -jnp.inf, dtype=jnp.float32)]
    )
    out = _sc_gather_combine(idx, w_pad, rc_cid_emb)
    return out.reshape(_B, _S, _EMB)


# speedup vs baseline: 1.1521x; 1.0270x over previous
import functools

import jax
import jax.numpy as jnp
from jax import lax
from jax.experimental import pallas as pl
from jax.experimental.pallas import tpu as pltpu
from jax.experimental.pallas import tpu_sc as plsc

_B, _S, _LVL, _EMB = 1024, 50, 8, 64
_LANES = 16
_NW = 32
_BPW = _B // _NW
_CROWS = _BPW * _LVL
_IDXR = _CROWS // 128


def _sc_body(idx_hbm, w_hbm, table_hbm, out_hbm, idx_v, rows_v, out_v, wv,
             gsem0, gsem1):
    info = plsc.get_sparse_core_info()
    nc = info.num_cores
    wid = lax.axis_index("s") * nc + lax.axis_index("c")
    b0 = wid * _BPW
    gsems = (gsem0, gsem1)

    pltpu.sync_copy(w_hbm, wv)
    w = wv[...]
    ws = [w[l] for l in range(_LVL)]
    m = ws[0]
    for l in range(1, _LVL):
        m = jnp.maximum(m, ws[l])
    e = jnp.exp(w - m)
    es = [e[l] for l in range(_LVL)]
    s_sum = es[0]
    for l in range(1, _LVL):
        s_sum = s_sum + es[l]
    alpha = e / s_sum
    a = [alpha[l] for l in range(_LVL)]

    iota = lax.broadcasted_iota(jnp.int32, (_LANES,), 0)

    def fire(s, buf):
        pltpu.sync_copy(idx_hbm.at[wid, s], idx_v.at[buf])
        for j in range(_IDXR):
            pltpu.async_copy(
                table_hbm.at[idx_v.at[buf, j]],
                rows_v.at[buf, pl.ds(j * 128, 128)],
                gsems[buf],
            )

    def drain(buf):
        for j in range(_IDXR):
            pltpu.make_async_copy(
                table_hbm.at[idx_v.at[buf, j]],
                rows_v.at[buf, pl.ds(j * 128, 128)],
                gsems[buf],
            ).wait()

    def combine(s, buf):
        def tok(bb, c):
            rbase = bb * _LVL
            col = iota * 0 + bb
            for j in range(_EMB // _LANES):
                sl = pl.ds(j * _LANES, _LANES)
                acc = a[0] * rows_v[buf, rbase, sl]
                for l in range(1, _LVL):
                    acc = acc + a[l] * rows_v[buf, rbase + l, sl]
                plsc.store_scatter(out_v, [j * _LANES + iota, col], acc)
            return c

        lax.fori_loop(0, _BPW, tok, 0)
        pltpu.sync_copy(out_v, out_hbm.at[s, :, pl.ds(b0, _BPW)])

    fire(0, 0)

    def pair(p, carry):
        s0 = 2 * p
        fire(s0 + 1, 1)
        drain(0)
        combine(s0, 0)

        @pl.when(s0 + 2 < _S)
        def _():
            fire(s0 + 2, 0)

        drain(1)
        combine(s0 + 1, 1)
        return carry

    lax.fori_loop(0, _S // 2, pair, 0)


@jax.jit
def _sc_gather_combine(idx, w_pad, table):
    run = functools.partial(
        pl.kernel,
        out_type=jax.ShapeDtypeStruct((_S, _EMB, _B), jnp.float32),
        mesh=plsc.VectorSubcoreMesh(core_axis_name="c", subcore_axis_name="s"),
        scratch_types=[
            pltpu.VMEM((2, _IDXR, 128), jnp.int32),
            pltpu.VMEM((2, _CROWS, _EMB), jnp.float32),
            pltpu.VMEM((_EMB, _BPW), jnp.float32),
            pltpu.VMEM((_LANES,), jnp.float32),
            pltpu.SemaphoreType.DMA,
            pltpu.SemaphoreType.DMA,
        ],
        compiler_params=pltpu.CompilerParams(
            use_tc_tiling_on_sc=False, needs_layout_passes=False
        ),
    )(_sc_body)
    return run(idx, w_pad, table)


def kernel(croutes, tailcs, rc_cid_emb, rc_weight):
    del tailcs
    idx = (
        croutes.reshape(_NW, _BPW, _S, _LVL)
        .transpose(0, 2, 1, 3)
        .reshape(_NW, _S, _IDXR, 128)
    )
    w_pad = jnp.concatenate(
        [rc_weight.astype(jnp.float32),
         jnp.full((_LANES - _LVL,), -jnp.inf, dtype=jnp.float32)]
    )
    out_phys = _sc_gather_combine(idx, w_pad, rc_cid_emb)
    return out_phys.transpose(2, 0, 1)
